# baseline (speedup 1.0000x reference)
import functools

import jax
import jax.numpy as jnp
from jax import lax
from jax.experimental import pallas as pl
from jax.experimental.pallas import tpu as pltpu
from jax.experimental.pallas import tpu_sc as plsc

N = 10000
E = 320000
F_IN = 128
HID = 64
C = 16
K = 10
ALPHA = 0.1

NS = 16
NC = 2
NW = NC * NS
N_PAD = 10240
RPT = N_PAD // NW
HALF = N_PAD // 2
CHUNK = 512
CPT = 20
CPT_A = CPT + 1
DUMMY = N + 8


def _mlp_body(x_ref, w1t_ref, b1_ref, w2t_ref, b2_ref, o_ref):
    h = jnp.maximum(
        jnp.dot(x_ref[...], w1t_ref[...], preferred_element_type=jnp.float32)
        + b1_ref[...],
        0.0,
    )
    o_ref[...] = (
        jnp.dot(h, w2t_ref[...], preferred_element_type=jnp.float32) + b2_ref[...]
    )


def _mlp(x, W1, b1, W2, b2):
    grid = (10,)
    bn = N // 10
    return pl.pallas_call(
        _mlp_body,
        grid=grid,
        in_specs=[
            pl.BlockSpec((bn, F_IN), lambda i: (i, 0)),
            pl.BlockSpec((F_IN, HID), lambda i: (0, 0)),
            pl.BlockSpec((1, HID), lambda i: (0, 0)),
            pl.BlockSpec((HID, C), lambda i: (0, 0)),
            pl.BlockSpec((1, C), lambda i: (0, 0)),
        ],
        out_specs=pl.BlockSpec((bn, C), lambda i: (i, 0)),
        out_shape=jax.ShapeDtypeStruct((N, C), jnp.float32),
    )(x, W1.T, b1[None, :], W2.T, b2[None, :])


def _logsoftmax_body(x_ref, o_ref):
    x = x_ref[...]
    m = jnp.max(x, axis=1, keepdims=True)
    s = x - m
    o_ref[...] = s - jnp.log(jnp.sum(jnp.exp(s), axis=1, keepdims=True))


def _logsoftmax(x):
    grid = (10,)
    bn = N // 10
    return pl.pallas_call(
        _logsoftmax_body,
        grid=grid,
        in_specs=[pl.BlockSpec((bn, C), lambda i: (i, 0))],
        out_specs=pl.BlockSpec((bn, C), lambda i: (i, 0)),
        out_shape=jax.ShapeDtypeStruct((N, C), jnp.float32),
    )(x)


def _rsqrt16(d):
    a = 1.0 / d
    x = jnp.full((C,), 0.04, jnp.float32)
    for _ in range(12):
        x = 0.5 * (x + a / x)
    return x


def _propagate_body(row_h, col_h, z_h, out_h, aggd_h,
                    row_v, col_v, gbuf0, gbuf1, zrow_v, dinvb_v,
                    outs_v, agg_v, sum_v, zeros_v, outs_sh, agg_sh,
                    gsem0, gsem1, ssem0, ssem1, xsem1, xsem2):
    cid = lax.axis_index("c")
    sid = lax.axis_index("s")
    wid = cid * NS + sid
    mwid = (1 - cid) * NS + sid
    base = pl.multiple_of(wid * RPT, 8)
    mbase = pl.multiple_of(mwid * RPT, 8)

    pltpu.sync_copy(row_h.at[wid], row_v)
    pltpu.sync_copy(col_h.at[wid], col_v)
    pltpu.sync_copy(z_h.at[pl.ds(base, RPT)], zrow_v)

    one16 = jnp.ones((C,), jnp.float32)
    zero16 = jnp.zeros((C,), jnp.float32)

    def fill_ones(i, c):
        gbuf0[i, :] = one16
        return c

    lax.fori_loop(0, CHUNK, fill_ones, 0)

    def fill_zeros(i, c):
        zeros_v[i, :] = zero16
        return c

    lax.fori_loop(0, CHUNK, fill_zeros, 0)

    pltpu.sync_copy(zeros_v.at[pl.ds(0, RPT)], agg_sh.at[pl.ds(base, RPT)])
    pltpu.sync_copy(zeros_v.at[pl.ds(0, RPT)], agg_sh.at[pl.ds(mbase, RPT)])
    plsc.subcore_barrier()

    def exchange(is_count, r):
        pltpu.sync_copy(agg_sh.at[pl.ds(base, RPT)], agg_v)
        pltpu.sync_copy(agg_sh.at[pl.ds(mbase, RPT)],
                        aggd_h.at[cid, pl.ds(mbase, RPT)])
        pltpu.sync_copy(zeros_v.at[pl.ds(0, RPT)], agg_sh.at[pl.ds(base, RPT)])
        pltpu.sync_copy(zeros_v.at[pl.ds(0, RPT)], agg_sh.at[pl.ds(mbase, RPT)])
        pl.semaphore_signal(xsem1, 1, core_index=1 - cid)
        pl.semaphore_wait(xsem1, 1)
        pltpu.sync_copy(aggd_h.at[1 - cid, pl.ds(base, RPT)], sum_v)

        if is_count:
            def body(i, cc):
                d = agg_v[i, :] + sum_v[i, :] + 1.0
                y = _rsqrt16(d)
                dinvb_v[i, :] = y
                outs_v[i, :] = zrow_v[i, :] * y
                return cc
        else:
            is_last = r == (K - 1)

            def body(i, cc):
                db = dinvb_v[i, :]
                newout = (1.0 - ALPHA) * db \
                    * (agg_v[i, :] + sum_v[i, :] + outs_v[i, :]) \
                    + ALPHA * zrow_v[i, :]
                outs_v[i, :] = jnp.where(is_last, newout, newout * db)
                return cc

        lax.fori_loop(0, RPT, body, 0)
        pltpu.sync_copy(outs_v, out_h.at[pl.ds(base, RPT)])
        pltpu.sync_copy(outs_v, outs_sh.at[pl.ds(base, RPT)])
        pl.semaphore_signal(xsem2, 1, core_index=1 - cid)
        pl.semaphore_wait(xsem2, 1)
        pltpu.sync_copy(out_h.at[pl.ds(mbase, RPT)],
                        outs_sh.at[pl.ds(mbase, RPT)])
        plsc.subcore_barrier()

    def count(i, c):
        pltpu.sync_copy(gbuf0, agg_sh.at[col_v.at[i]], add=True)
        return c

    lax.fori_loop(0, CPT, count, 0)
    plsc.subcore_barrier()
    exchange(True, 0)

    zchunk = zeros_v.at[pl.ds(0, CHUNK)]
    dummy_idx = row_v.at[CPT]

    def round_body(r, c):
        pltpu.async_copy(zchunk, agg_sh.at[dummy_idx], ssem1, add=True)
        pltpu.async_copy(outs_sh.at[row_v.at[0]], gbuf0, gsem0)

        def edge2(j, cc):
            i0 = 2 * j
            pltpu.make_async_copy(outs_sh.at[row_v.at[i0]], gbuf0,
                                  gsem0).wait()
            pltpu.make_async_copy(zchunk, agg_sh.at[dummy_idx], ssem1).wait()
            pltpu.async_copy(outs_sh.at[row_v.at[i0 + 1]], gbuf1, gsem1)
            pltpu.async_copy(gbuf0, agg_sh.at[col_v.at[i0]], ssem0, add=True)
            pltpu.make_async_copy(outs_sh.at[row_v.at[i0]], gbuf1,
                                  gsem1).wait()
            pltpu.make_async_copy(zchunk, agg_sh.at[dummy_idx], ssem0).wait()
            pltpu.async_copy(outs_sh.at[row_v.at[i0 + 2]], gbuf0, gsem0)
            pltpu.async_copy(gbuf1, agg_sh.at[col_v.at[i0 + 1]], ssem1,
                             add=True)
            return cc

        lax.fori_loop(0, CPT // 2, edge2, 0)
        pltpu.make_async_copy(outs_sh.at[row_v.at[0]], gbuf0, gsem0).wait()
        pltpu.make_async_copy(zchunk, agg_sh.at[dummy_idx], ssem1).wait()
        plsc.subcore_barrier()
        exchange(False, r)
        return c

    lax.fori_loop(0, K, round_body, 0)


def _propagate(row_t, col_t, z_pad):
    mesh = plsc.VectorSubcoreMesh(
        core_axis_name="c", subcore_axis_name="s", num_cores=NC
    )
    out, _ = pl.kernel(
        _propagate_body,
        out_type=[
            jax.ShapeDtypeStruct((N_PAD, C), jnp.float32),
            jax.ShapeDtypeStruct((NC, N_PAD, C), jnp.float32),
        ],
        mesh=mesh,
        compiler_params=pltpu.CompilerParams(use_tc_tiling_on_sc=False),
        scratch_types=[
            pltpu.VMEM((CPT_A, CHUNK), jnp.int32),
            pltpu.VMEM((CPT_A, CHUNK), jnp.int32),
            pltpu.VMEM((CHUNK, C), jnp.float32),
            pltpu.VMEM((CHUNK, C), jnp.float32),
            pltpu.VMEM((RPT, C), jnp.float32),
            pltpu.VMEM((RPT, C), jnp.float32),
            pltpu.VMEM((RPT, C), jnp.float32),
            pltpu.VMEM((RPT, C), jnp.float32),
            pltpu.VMEM((RPT, C), jnp.float32),
            pltpu.VMEM((CHUNK, C), jnp.float32),
            pltpu.VMEM_SHARED((N_PAD, C), jnp.float32),
            pltpu.VMEM_SHARED((N_PAD, C), jnp.float32),
            pltpu.SemaphoreType.DMA,
            pltpu.SemaphoreType.DMA,
            pltpu.SemaphoreType.DMA,
            pltpu.SemaphoreType.DMA,
            pltpu.SemaphoreType.REGULAR,
            pltpu.SemaphoreType.REGULAR,
        ],
    )(row_t, col_t, z_pad)
    return out


@jax.jit
def kernel(x, edge_index, W1, b1, W2, b2):
    z = _mlp(x, W1, b1, W2, b2)

    pad = jnp.full((NW * CPT * CHUNK - E,), DUMMY, jnp.int32)
    dchunk = jnp.full((NW, 1, CHUNK), DUMMY, jnp.int32)

    def _layout(e):
        t = jnp.concatenate([e, pad]).reshape(NW, CPT, CHUNK)
        return jnp.concatenate([t, dchunk], axis=1)

    row_t = _layout(edge_index[0])
    col_t = _layout(edge_index[1])
    z_pad = jnp.pad(z, ((0, N_PAD - N), (0, 0)))

    out = _propagate(row_t, col_t, z_pad)
    return _logsoftmax(out[:N])


# speedup vs baseline: 1.2138x; 1.2138x over previous
import functools

import jax
import jax.numpy as jnp
from jax import lax
from jax.experimental import pallas as pl
from jax.experimental.pallas import tpu as pltpu
from jax.experimental.pallas import tpu_sc as plsc

N = 10000
E = 320000
F_IN = 128
HID = 64
C = 16
K = 10
ALPHA = 0.1

NS = 16
NC = 2
NW = NC * NS
N_PAD = 10240
RPT = N_PAD // NW
HALF = N_PAD // 2
CHUNK = 128
CPT = 80
CPT_A = CPT + 1
DUMMY = N + 8


def _mlp_body(x_ref, w1t_ref, b1_ref, w2t_ref, b2_ref, o_ref):
    h = jnp.maximum(
        jnp.dot(x_ref[...], w1t_ref[...], preferred_element_type=jnp.float32)
        + b1_ref[...],
        0.0,
    )
    o_ref[...] = (
        jnp.dot(h, w2t_ref[...], preferred_element_type=jnp.float32) + b2_ref[...]
    )


def _mlp(x, W1, b1, W2, b2):
    grid = (10,)
    bn = N // 10
    return pl.pallas_call(
        _mlp_body,
        grid=grid,
        in_specs=[
            pl.BlockSpec((bn, F_IN), lambda i: (i, 0)),
            pl.BlockSpec((F_IN, HID), lambda i: (0, 0)),
            pl.BlockSpec((1, HID), lambda i: (0, 0)),
            pl.BlockSpec((HID, C), lambda i: (0, 0)),
            pl.BlockSpec((1, C), lambda i: (0, 0)),
        ],
        out_specs=pl.BlockSpec((bn, C), lambda i: (i, 0)),
        out_shape=jax.ShapeDtypeStruct((N, C), jnp.float32),
    )(x, W1.T, b1[None, :], W2.T, b2[None, :])


def _logsoftmax_body(x_ref, o_ref):
    x = x_ref[...]
    m = jnp.max(x, axis=1, keepdims=True)
    s = x - m
    o_ref[...] = s - jnp.log(jnp.sum(jnp.exp(s), axis=1, keepdims=True))


def _logsoftmax(x):
    grid = (10,)
    bn = N // 10
    return pl.pallas_call(
        _logsoftmax_body,
        grid=grid,
        in_specs=[pl.BlockSpec((bn, C), lambda i: (i, 0))],
        out_specs=pl.BlockSpec((bn, C), lambda i: (i, 0)),
        out_shape=jax.ShapeDtypeStruct((N, C), jnp.float32),
    )(x)


def _rsqrt16(d):
    a = 1.0 / d
    x = jnp.full((C,), 0.04, jnp.float32)
    for _ in range(12):
        x = 0.5 * (x + a / x)
    return x


def _propagate_body(row_h, col_h, z_h, out_h, aggd_h,
                    row_v, col_v, gbuf0, gbuf1, zrow_v, dinvb_v,
                    outs_v, agg_v, sum_v, zeros_v, outs_sh, agg_sh,
                    gsem0, gsem1, ssem0, ssem1, xsem1, xsem2):
    cid = lax.axis_index("c")
    sid = lax.axis_index("s")
    wid = cid * NS + sid
    mwid = (1 - cid) * NS + sid
    base = pl.multiple_of(wid * RPT, 8)
    mbase = pl.multiple_of(mwid * RPT, 8)

    pltpu.sync_copy(row_h.at[wid], row_v)
    pltpu.sync_copy(col_h.at[wid], col_v)
    pltpu.sync_copy(z_h.at[pl.ds(base, RPT)], zrow_v)

    one16 = jnp.ones((C,), jnp.float32)
    zero16 = jnp.zeros((C,), jnp.float32)

    def fill_ones(i, c):
        gbuf0[i, :] = one16
        return c

    lax.fori_loop(0, CHUNK, fill_ones, 0)

    def fill_zeros(i, c):
        zeros_v[i, :] = zero16
        return c

    lax.fori_loop(0, RPT, fill_zeros, 0)

    pltpu.sync_copy(zeros_v, agg_sh.at[pl.ds(base, RPT)])
    pltpu.sync_copy(zeros_v, agg_sh.at[pl.ds(mbase, RPT)])
    plsc.subcore_barrier()

    def exchange(is_count, r):
        pltpu.sync_copy(agg_sh.at[pl.ds(base, RPT)], agg_v)
        pltpu.sync_copy(agg_sh.at[pl.ds(mbase, RPT)],
                        aggd_h.at[cid, pl.ds(mbase, RPT)])
        pltpu.sync_copy(zeros_v, agg_sh.at[pl.ds(base, RPT)])
        pltpu.sync_copy(zeros_v, agg_sh.at[pl.ds(mbase, RPT)])
        pl.semaphore_signal(xsem1, 1, core_index=1 - cid)
        pl.semaphore_wait(xsem1, 1)
        pltpu.sync_copy(aggd_h.at[1 - cid, pl.ds(base, RPT)], sum_v)

        if is_count:
            def body(i, cc):
                d = agg_v[i, :] + sum_v[i, :] + 1.0
                y = _rsqrt16(d)
                dinvb_v[i, :] = y
                outs_v[i, :] = zrow_v[i, :] * y
                return cc
        else:
            is_last = r == (K - 1)

            def body(i, cc):
                db = dinvb_v[i, :]
                newout = (1.0 - ALPHA) * db \
                    * (agg_v[i, :] + sum_v[i, :] + outs_v[i, :]) \
                    + ALPHA * zrow_v[i, :]
                outs_v[i, :] = jnp.where(is_last, newout, newout * db)
                return cc

        lax.fori_loop(0, RPT, body, 0)
        pltpu.sync_copy(outs_v, out_h.at[pl.ds(base, RPT)])
        pltpu.sync_copy(outs_v, outs_sh.at[pl.ds(base, RPT)])
        pl.semaphore_signal(xsem2, 1, core_index=1 - cid)
        pl.semaphore_wait(xsem2, 1)
        pltpu.sync_copy(out_h.at[pl.ds(mbase, RPT)],
                        outs_sh.at[pl.ds(mbase, RPT)])
        plsc.subcore_barrier()

    def count(i, c):
        pltpu.sync_copy(gbuf0, agg_sh.at[col_v.at[i]], add=True)
        return c

    lax.fori_loop(0, CPT, count, 0)
    plsc.subcore_barrier()
    exchange(True, 0)

    zchunk = zeros_v.at[pl.ds(0, CHUNK)]
    dummy_idx = row_v.at[CPT]

    def round_body(r, c):
        pltpu.async_copy(zchunk, agg_sh.at[dummy_idx], ssem1, add=True)
        pltpu.async_copy(outs_sh.at[row_v.at[0]], gbuf0, gsem0)

        def edge2(j, cc):
            i0 = 2 * j
            pltpu.make_async_copy(outs_sh.at[row_v.at[i0]], gbuf0,
                                  gsem0).wait()
            pltpu.make_async_copy(zchunk, agg_sh.at[dummy_idx], ssem1).wait()
            pltpu.async_copy(outs_sh.at[row_v.at[i0 + 1]], gbuf1, gsem1)
            pltpu.async_copy(gbuf0, agg_sh.at[col_v.at[i0]], ssem0, add=True)
            pltpu.make_async_copy(outs_sh.at[row_v.at[i0]], gbuf1,
                                  gsem1).wait()
            pltpu.make_async_copy(zchunk, agg_sh.at[dummy_idx], ssem0).wait()
            pltpu.async_copy(outs_sh.at[row_v.at[i0 + 2]], gbuf0, gsem0)
            pltpu.async_copy(gbuf1, agg_sh.at[col_v.at[i0 + 1]], ssem1,
                             add=True)
            return cc

        lax.fori_loop(0, CPT // 2, edge2, 0)
        pltpu.make_async_copy(outs_sh.at[row_v.at[0]], gbuf0, gsem0).wait()
        pltpu.make_async_copy(zchunk, agg_sh.at[dummy_idx], ssem1).wait()
        plsc.subcore_barrier()
        exchange(False, r)
        return c

    lax.fori_loop(0, K, round_body, 0)


def _propagate(row_t, col_t, z_pad):
    mesh = plsc.VectorSubcoreMesh(
        core_axis_name="c", subcore_axis_name="s", num_cores=NC
    )
    out, _ = pl.kernel(
        _propagate_body,
        out_type=[
            jax.ShapeDtypeStruct((N_PAD, C), jnp.float32),
            jax.ShapeDtypeStruct((NC, N_PAD, C), jnp.float32),
        ],
        mesh=mesh,
        compiler_params=pltpu.CompilerParams(use_tc_tiling_on_sc=False),
        scratch_types=[
            pltpu.VMEM((CPT_A, CHUNK), jnp.int32),
            pltpu.VMEM((CPT_A, CHUNK), jnp.int32),
            pltpu.VMEM((CHUNK, C), jnp.float32),
            pltpu.VMEM((CHUNK, C), jnp.float32),
            pltpu.VMEM((RPT, C), jnp.float32),
            pltpu.VMEM((RPT, C), jnp.float32),
            pltpu.VMEM((RPT, C), jnp.float32),
            pltpu.VMEM((RPT, C), jnp.float32),
            pltpu.VMEM((RPT, C), jnp.float32),
            pltpu.VMEM((RPT, C), jnp.float32),
            pltpu.VMEM_SHARED((N_PAD, C), jnp.float32),
            pltpu.VMEM_SHARED((N_PAD, C), jnp.float32),
            pltpu.SemaphoreType.DMA,
            pltpu.SemaphoreType.DMA,
            pltpu.SemaphoreType.DMA,
            pltpu.SemaphoreType.DMA,
            pltpu.SemaphoreType.REGULAR,
            pltpu.SemaphoreType.REGULAR,
        ],
    )(row_t, col_t, z_pad)
    return out


@jax.jit
def kernel(x, edge_index, W1, b1, W2, b2):
    z = _mlp(x, W1, b1, W2, b2)

    pad = jnp.full((NW * CPT * CHUNK - E,), DUMMY, jnp.int32)
    dchunk = jnp.full((NW, 1, CHUNK), DUMMY, jnp.int32)

    def _layout(e):
        t = jnp.concatenate([e, pad]).reshape(NW, CPT, CHUNK)
        return jnp.concatenate([t, dchunk], axis=1)

    row_t = _layout(edge_index[0])
    col_t = _layout(edge_index[1])
    z_pad = jnp.pad(z, ((0, N_PAD - N), (0, 0)))

    out = _propagate(row_t, col_t, z_pad)
    return _logsoftmax(out[:N])
